# baseline (speedup 1.0000x reference)
import functools

import jax
import jax.numpy as jnp
from jax import lax
from jax.experimental import pallas as pl
from jax.experimental.pallas import tpu as pltpu
from jax.experimental.pallas import tpu_sc as plsc

_T = 2048
_H = 2048
_HL = _H // 128
_DFF = 768
_E = 8
_K = 2
_P = _T * _K
_EPAD = 128
_BT = 256
_BM = 256
_NB = (_P + _E * (_BM - 1)) // _BM + 1
_NBBM = _NB * _BM
_NC = 2
_NS = 16
_NW = _NC * _NS

_DROWS = _NBBM // _NW
_DCH = 48
_CROWS = _T // _NW
_CCH = 16


def _router_body(x_ref, gwt_ref, sel_ref, w_ref):
    x = x_ref[...]
    logits = lax.dot_general(
        x, gwt_ref[...], (((1,), (0,)), ((), ())),
        preferred_element_type=jnp.float32)
    col = lax.broadcasted_iota(jnp.int32, (_BT, _EPAD), 1)
    valid = col < _E
    logits = jnp.where(valid, logits, jnp.float32(-1e30))
    m = jnp.max(logits, axis=1, keepdims=True)
    p = jnp.exp(logits - m)
    p = p / jnp.sum(p, axis=1, keepdims=True)
    p = jnp.where(valid, p, -1.0)
    m1 = jnp.max(p, axis=1, keepdims=True)
    i1 = jnp.min(jnp.where(p == m1, col, _EPAD), axis=1, keepdims=True)
    p2 = jnp.where(col == i1, -1.0, p)
    m2 = jnp.max(p2, axis=1, keepdims=True)
    i2 = jnp.min(jnp.where(p2 == m2, col, _EPAD), axis=1, keepdims=True)
    denom = m1 + m2
    w1 = m1 / denom
    w2 = m2 / denom
    sel_ref[...] = jnp.where(col == 0, i1, 0) + jnp.where(col == 1, i2, 0)
    w_ref[...] = (jnp.where(col == 0, w1, 0.0) + jnp.where(col == 1, w2, 0.0))


def _router(x, gate_w):
    gwt = jnp.zeros((_H, _EPAD), jnp.float32).at[:, :_E].set(gate_w.T)
    return pl.pallas_call(
        _router_body,
        grid=(_T // _BT,),
        in_specs=[
            pl.BlockSpec((_BT, _H), lambda i: (i, 0)),
            pl.BlockSpec((_H, _EPAD), lambda i: (0, 0)),
        ],
        out_specs=[
            pl.BlockSpec((_BT, _EPAD), lambda i: (i, 0)),
            pl.BlockSpec((_BT, _EPAD), lambda i: (i, 0)),
        ],
        out_shape=[
            jax.ShapeDtypeStruct((_T, _EPAD), jnp.int32),
            jax.ShapeDtypeStruct((_T, _EPAD), jnp.float32),
        ],
    )(x, gwt)


def _route_indices(sel, wts):
    e_flat = sel.reshape(-1)
    w_flat = wts.reshape(-1)
    oh = (e_flat[:, None] == jnp.arange(_E, dtype=jnp.int32)[None, :])
    oh = oh.astype(jnp.int32)
    csum = jnp.cumsum(oh, axis=0)
    rank = jnp.sum((csum - oh) * oh, axis=1)
    counts = csum[-1]
    padded = ((counts + _BM - 1) // _BM) * _BM
    ends = jnp.cumsum(padded)
    base = ends - padded
    pos = base[e_flat] + rank
    tok = jnp.arange(_P, dtype=jnp.int32) // _K
    gidx = jnp.zeros((_NBBM,), jnp.int32).at[pos].set(tok)
    rw = jnp.zeros((_NBBM,), jnp.float32).at[pos].set(w_flat)
    starts = jnp.arange(_NB, dtype=jnp.int32) * _BM
    block_expert = jnp.minimum(
        jnp.searchsorted(ends, starts, side='right'), _E - 1).astype(jnp.int32)
    return gidx, rw, block_expert, pos[0::2], pos[1::2]


@functools.cache
def _make_dispatch():
    mesh = plsc.VectorSubcoreMesh(core_axis_name="c", subcore_axis_name="s")

    @functools.partial(
        pl.kernel,
        out_type=jax.ShapeDtypeStruct((_NBBM, _H // 2), jnp.int32),
        mesh=mesh,
        scratch_types=[
            pltpu.VMEM((_DROWS,), jnp.int32),
            pltpu.VMEM((_DCH, _H // 2), jnp.int32),
            pltpu.SemaphoreType.DMA,
        ],
    )
    def dispatch(x_hbm, gidx_hbm, out_hbm, idx_v, rows_v, sem):
        wid = lax.axis_index("s") * _NC + lax.axis_index("c")
        base = wid * _DROWS
        pltpu.sync_copy(gidx_hbm.at[pl.ds(base, _DROWS)], idx_v)
        for c in range(_DROWS // _DCH):
            pltpu.async_copy(
                x_hbm.at[idx_v.at[pl.ds(c * _DCH, _DCH)]], rows_v, sem).wait()
            pltpu.sync_copy(rows_v, out_hbm.at[pl.ds(base + c * _DCH, _DCH)])

    return dispatch


def _dispatch(x3, gidx):
    return _make_dispatch()(x3, gidx)


def _ffn_body(be_ref, xg_ref, w_ref, gp_ref, up_ref, dp_ref, out_ref):
    xb = xg_ref[...]
    g = lax.dot_general(
        xb, gp_ref[0], (((1,), (0,)), ((), ())),
        preferred_element_type=jnp.float32)
    u = lax.dot_general(
        xb, up_ref[0], (((1,), (0,)), ((), ())),
        preferred_element_type=jnp.float32)
    h = (g * lax.logistic(g) * u).astype(jnp.bfloat16)
    y = lax.dot_general(
        h, dp_ref[0], (((1,), (0,)), ((), ())),
        preferred_element_type=jnp.float32)
    out_ref[...] = y * w_ref[:, 0:1]


def _ffn(block_expert, xg, rww, gpt, upt, dpt):
    grid_spec = pltpu.PrefetchScalarGridSpec(
        num_scalar_prefetch=1,
        grid=(_NB,),
        in_specs=[
            pl.BlockSpec((_BM, _H), lambda b, be: (b, 0)),
            pl.BlockSpec((_BM, 128), lambda b, be: (b, 0)),
            pl.BlockSpec((1, _H, _DFF), lambda b, be: (be[b], 0, 0)),
            pl.BlockSpec((1, _H, _DFF), lambda b, be: (be[b], 0, 0)),
            pl.BlockSpec((1, _DFF, _H), lambda b, be: (be[b], 0, 0)),
        ],
        out_specs=pl.BlockSpec((_BM, _H), lambda b, be: (b, 0)),
    )
    return pl.pallas_call(
        _ffn_body,
        grid_spec=grid_spec,
        out_shape=jax.ShapeDtypeStruct((_NBBM, _H), jnp.float32),
    )(block_expert, xg, rww, gpt, upt, dpt)


@functools.cache
def _make_combine():
    mesh = plsc.VectorSubcoreMesh(core_axis_name="c", subcore_axis_name="s")

    @functools.partial(
        pl.kernel,
        out_type=jax.ShapeDtypeStruct((_T, _H), jnp.float32),
        mesh=mesh,
        scratch_types=[
            pltpu.VMEM((_CROWS,), jnp.int32),
            pltpu.VMEM((_CROWS,), jnp.int32),
            pltpu.VMEM((_CCH, _H), jnp.float32),
            pltpu.VMEM((_CCH, _H), jnp.float32),
            pltpu.SemaphoreType.DMA,
            pltpu.SemaphoreType.DMA,
        ],
    )
    def combine(yg_hbm, posa_hbm, posb_hbm, out_hbm, ia_v, ib_v, ba_v, bb_v,
                sa, sb):
        wid = lax.axis_index("s") * _NC + lax.axis_index("c")
        base = wid * _CROWS
        pltpu.sync_copy(posa_hbm.at[pl.ds(base, _CROWS)], ia_v)
        pltpu.sync_copy(posb_hbm.at[pl.ds(base, _CROWS)], ib_v)
        for c in range(_CROWS // _CCH):
            cpa = pltpu.async_copy(
                yg_hbm.at[ia_v.at[pl.ds(c * _CCH, _CCH)]], ba_v, sa)
            cpb = pltpu.async_copy(
                yg_hbm.at[ib_v.at[pl.ds(c * _CCH, _CCH)]], bb_v, sb)
            cpa.wait()
            cpb.wait()

            def _add(i, _):
                r = i // (_H // 16)
                off = (i - r * (_H // 16)) * 16
                ba_v[r, pl.ds(off, 16)] = (
                    ba_v[r, pl.ds(off, 16)] + bb_v[r, pl.ds(off, 16)])
                return 0

            lax.fori_loop(0, _CCH * (_H // 16), _add, 0)
            pltpu.sync_copy(ba_v, out_hbm.at[pl.ds(base + c * _CCH, _CCH)])

    return combine


def _combine(yg, pos_a, pos_b):
    return _make_combine()(yg, pos_a, pos_b)


@jax.jit
def kernel(hidden_states, gate_w, gate_proj_w, up_proj_w, down_proj_w):
    B, S, H = hidden_states.shape
    x = hidden_states.reshape(-1, H)
    sel128, w128 = _router(x, gate_w)
    gidx, rw, block_expert, pos_a, pos_b = _route_indices(
        sel128[:, :_K], w128[:, :_K])
    xi = jax.lax.bitcast_convert_type(
        x.astype(jnp.bfloat16).reshape(_T, _H // 2, 2), jnp.int32)
    xg = jax.lax.bitcast_convert_type(
        _dispatch(xi, gidx), jnp.bfloat16).reshape(_NBBM, _H)
    rww = jnp.broadcast_to(rw[:, None], (_NBBM, 128))
    gpt = jnp.swapaxes(gate_proj_w, 1, 2).astype(jnp.bfloat16)
    upt = jnp.swapaxes(up_proj_w, 1, 2).astype(jnp.bfloat16)
    dpt = jnp.swapaxes(down_proj_w, 1, 2).astype(jnp.bfloat16)
    yg = _ffn(block_expert, xg, rww, gpt, upt, dpt)
    out = _combine(yg, pos_a, pos_b)
    return out.reshape(B, S, H)
